# Initial kernel scaffold; baseline (speedup 1.0000x reference)
import functools

import jax
import jax.numpy as jnp
from jax import lax
from jax.experimental import pallas as pl
from jax.experimental.pallas import tpu as pltpu
from jax.experimental.pallas import tpu_sc as plsc

N = 10000
D = 128
E = 320000

NC = 2
NS = 16
NW = NC * NS
EPW = E // NW
K = 100
C = EPW // K
RPT = N // NS
DEGW = 16

RB = 2000

_mesh = plsc.VectorSubcoreMesh(
    core_axis_name="c", subcore_axis_name="s", num_cores=NC, num_subcores=NS)


def _deg_body(dst_hbm, ones_hbm, zeros_hbm, deg_out, dst_v, ones_v, deg_sh):
    cid = lax.axis_index("c")
    sid = lax.axis_index("s")
    wid = sid * NC + cid
    pltpu.sync_copy(zeros_hbm.at[pl.ds(sid * RPT, RPT)],
                    deg_sh.at[pl.ds(sid * RPT, RPT)])
    pltpu.sync_copy(dst_hbm.at[wid], dst_v)
    pltpu.sync_copy(ones_hbm, ones_v)
    plsc.subcore_barrier()

    def chunk(j, carry):
        pltpu.sync_copy(ones_v, deg_sh.at[dst_v.at[j]], add=True)
        return carry

    lax.fori_loop(0, C, chunk, 0)
    plsc.subcore_barrier()
    pltpu.sync_copy(deg_sh.at[pl.ds(sid * RPT, RPT)],
                    deg_out.at[cid, pl.ds(sid * RPT, RPT)])


_deg_call = pl.kernel(
    _deg_body,
    out_type=jax.ShapeDtypeStruct((NC, N, DEGW), jnp.float32),
    mesh=_mesh,
    scratch_types=[
        pltpu.VMEM((C, K), jnp.int32),
        pltpu.VMEM((K, DEGW), jnp.float32),
        pltpu.VMEM_SHARED((N, DEGW), jnp.float32),
    ],
)


def _scat_body(g_hbm, src_hbm, dst_hbm, zeros_hbm, s_out,
               src_v, dst_v, rows_v, s_sh, sem):
    cid = lax.axis_index("c")
    sid = lax.axis_index("s")
    wid = sid * NC + cid
    pltpu.sync_copy(zeros_hbm.at[pl.ds(sid * RPT, RPT)],
                    s_sh.at[pl.ds(sid * RPT, RPT)])
    pltpu.sync_copy(src_hbm.at[wid], src_v)
    pltpu.sync_copy(dst_hbm.at[wid], dst_v)
    plsc.subcore_barrier()

    def chunk(j, carry):
        pltpu.async_copy(g_hbm.at[src_v.at[j]], rows_v, sem).wait()
        pltpu.sync_copy(rows_v, s_sh.at[dst_v.at[j]], add=True)
        return carry

    lax.fori_loop(0, C, chunk, 0)
    plsc.subcore_barrier()
    pltpu.sync_copy(s_sh.at[pl.ds(sid * RPT, RPT)],
                    s_out.at[cid, pl.ds(sid * RPT, RPT)])


_scat_call = pl.kernel(
    _scat_body,
    out_type=jax.ShapeDtypeStruct((NC, N, D), jnp.float32),
    mesh=_mesh,
    scratch_types=[
        pltpu.VMEM((C, K), jnp.int32),
        pltpu.VMEM((C, K), jnp.int32),
        pltpu.VMEM((K, D), jnp.float32),
        pltpu.VMEM_SHARED((N, D), jnp.float32),
        pltpu.SemaphoreType.DMA,
    ],
)


def _mm_body(x_ref, w_ref, o_ref):
    o_ref[...] = jnp.dot(x_ref[...], w_ref[...],
                         preferred_element_type=jnp.float32)


def _matmul(x, w):
    return pl.pallas_call(
        _mm_body,
        grid=(N // RB,),
        in_specs=[pl.BlockSpec((RB, D), lambda i: (i, 0)),
                  pl.BlockSpec((D, D), lambda i: (0, 0))],
        out_specs=pl.BlockSpec((RB, D), lambda i: (i, 0)),
        out_shape=jax.ShapeDtypeStruct((N, D), jnp.float32),
    )(x, w)


def _dinv_of(deg_ref):
    d0 = deg_ref[0][:, 0:1]
    d1 = deg_ref[1][:, 0:1]
    return lax.rsqrt(d0 + d1 + 1.0)


def _scale_body(deg_ref, h_ref, g_ref):
    g_ref[...] = h_ref[...] * _dinv_of(deg_ref)


def _scale(deg_parts, h):
    return pl.pallas_call(
        _scale_body,
        grid=(N // RB,),
        in_specs=[pl.BlockSpec((NC, RB, DEGW), lambda i: (0, i, 0)),
                  pl.BlockSpec((RB, D), lambda i: (i, 0))],
        out_specs=pl.BlockSpec((RB, D), lambda i: (i, 0)),
        out_shape=jax.ShapeDtypeStruct((N, D), jnp.float32),
    )(deg_parts, h)


def _combine_mm_body(s_ref, g_ref, deg_ref, b_ref, w_ref, o_ref):
    dinv = _dinv_of(deg_ref)
    pre = (s_ref[0] + s_ref[1] + g_ref[...]) * dinv + b_ref[...]
    act = jnp.maximum(pre, 0.0)
    o_ref[...] = jnp.dot(act, w_ref[...],
                         preferred_element_type=jnp.float32) * dinv


def _combine_mm(s_parts, g, deg_parts, b_row, w):
    return pl.pallas_call(
        _combine_mm_body,
        grid=(N // RB,),
        in_specs=[pl.BlockSpec((NC, RB, D), lambda i: (0, i, 0)),
                  pl.BlockSpec((RB, D), lambda i: (i, 0)),
                  pl.BlockSpec((NC, RB, DEGW), lambda i: (0, i, 0)),
                  pl.BlockSpec((1, D), lambda i: (0, 0)),
                  pl.BlockSpec((D, D), lambda i: (0, 0))],
        out_specs=pl.BlockSpec((RB, D), lambda i: (i, 0)),
        out_shape=jax.ShapeDtypeStruct((N, D), jnp.float32),
    )(s_parts, g, deg_parts, b_row, w)


def _combine_body(s_ref, g_ref, deg_ref, b_ref, o_ref):
    dinv = _dinv_of(deg_ref)
    pre = (s_ref[0] + s_ref[1] + g_ref[...]) * dinv + b_ref[...]
    o_ref[...] = jnp.maximum(pre, 0.0)


def _combine(s_parts, g, deg_parts, b_row):
    return pl.pallas_call(
        _combine_body,
        grid=(N // RB,),
        in_specs=[pl.BlockSpec((NC, RB, D), lambda i: (0, i, 0)),
                  pl.BlockSpec((RB, D), lambda i: (i, 0)),
                  pl.BlockSpec((NC, RB, DEGW), lambda i: (0, i, 0)),
                  pl.BlockSpec((1, D), lambda i: (0, 0))],
        out_specs=pl.BlockSpec((RB, D), lambda i: (i, 0)),
        out_shape=jax.ShapeDtypeStruct((N, D), jnp.float32),
    )(s_parts, g, deg_parts, b_row)


def kernel(x, edge_index, W1, b1, W2, b2):
    src = edge_index[0].reshape(NW, C, K)
    dst = edge_index[1].reshape(NW, C, K)
    ones_deg = jnp.ones((K, DEGW), jnp.float32)
    zeros_deg = jnp.zeros((N, DEGW), jnp.float32)
    zeros_rows = jnp.zeros((N, D), jnp.float32)

    deg_parts = _deg_call(dst, ones_deg, zeros_deg)
    h1 = _matmul(x, W1)
    g1 = _scale(deg_parts, h1)
    s_parts = _scat_call(g1, src, dst, zeros_rows)
    g2 = _combine_mm(s_parts, g1, deg_parts, b1.reshape(1, D), W2)
    t_parts = _scat_call(g2, src, dst, zeros_rows)
    return _combine(t_parts, g2, deg_parts, b2.reshape(1, D))


# speedup vs baseline: 17.4283x; 17.4283x over previous
import functools

import jax
import jax.numpy as jnp
from jax import lax
from jax.experimental import pallas as pl
from jax.experimental.pallas import tpu as pltpu
from jax.experimental.pallas import tpu_sc as plsc

N = 10000
D = 128
E = 320000

NC = 2
NS = 16
NW = NC * NS
EPW = E // NW
K = 100
C = EPW // K
N_PAD = 10240
RPT = N_PAD // NS
DEGW = 16

RB = 2000

_mesh = plsc.VectorSubcoreMesh(
    core_axis_name="c", subcore_axis_name="s", num_cores=NC, num_subcores=NS)


def _deg_body(dst_hbm, ones_hbm, zeros_hbm, deg_out, dst_v, ones_v, deg_sh):
    cid = lax.axis_index("c")
    sid = lax.axis_index("s")
    wid = sid * NC + cid
    pltpu.sync_copy(zeros_hbm.at[pl.ds(sid * RPT, RPT)],
                    deg_sh.at[pl.ds(sid * RPT, RPT)])
    pltpu.sync_copy(dst_hbm.at[wid], dst_v)
    pltpu.sync_copy(ones_hbm, ones_v)
    plsc.subcore_barrier()

    def chunk(j, carry):
        pltpu.sync_copy(ones_v, deg_sh.at[dst_v.at[j]], add=True)
        return carry

    lax.fori_loop(0, C, chunk, 0)
    plsc.subcore_barrier()
    pltpu.sync_copy(deg_sh.at[pl.ds(sid * RPT, RPT)],
                    deg_out.at[cid, pl.ds(sid * RPT, RPT)])


_deg_call = pl.kernel(
    _deg_body,
    out_type=jax.ShapeDtypeStruct((NC, N_PAD, DEGW), jnp.float32),
    mesh=_mesh,
    scratch_types=[
        pltpu.VMEM((C, K), jnp.int32),
        pltpu.VMEM((K, DEGW), jnp.float32),
        pltpu.VMEM_SHARED((N_PAD, DEGW), jnp.float32),
    ],
    compiler_params=pltpu.CompilerParams(use_tc_tiling_on_sc=False),
)


def _scat_body(g_hbm, src_hbm, dst_hbm, zeros_hbm, s_out,
               src_v, dst_v, rows_v, s_sh, sem):
    cid = lax.axis_index("c")
    sid = lax.axis_index("s")
    wid = sid * NC + cid
    pltpu.sync_copy(zeros_hbm.at[pl.ds(sid * RPT, RPT)],
                    s_sh.at[pl.ds(sid * RPT, RPT)])
    pltpu.sync_copy(src_hbm.at[wid], src_v)
    pltpu.sync_copy(dst_hbm.at[wid], dst_v)
    plsc.subcore_barrier()

    def chunk(j, carry):
        pltpu.async_copy(g_hbm.at[src_v.at[j]], rows_v, sem).wait()
        pltpu.sync_copy(rows_v, s_sh.at[dst_v.at[j]], add=True)
        return carry

    lax.fori_loop(0, C, chunk, 0)
    plsc.subcore_barrier()
    pltpu.sync_copy(s_sh.at[pl.ds(sid * RPT, RPT)],
                    s_out.at[cid, pl.ds(sid * RPT, RPT)])


_scat_call = pl.kernel(
    _scat_body,
    out_type=jax.ShapeDtypeStruct((NC, N_PAD, D), jnp.float32),
    mesh=_mesh,
    scratch_types=[
        pltpu.VMEM((C, K), jnp.int32),
        pltpu.VMEM((C, K), jnp.int32),
        pltpu.VMEM((K, D), jnp.float32),
        pltpu.VMEM_SHARED((N_PAD, D), jnp.float32),
        pltpu.SemaphoreType.DMA,
    ],
)


def _mm_body(x_ref, w_ref, o_ref):
    o_ref[...] = jnp.dot(x_ref[...], w_ref[...],
                         preferred_element_type=jnp.float32)


def _matmul(x, w):
    return pl.pallas_call(
        _mm_body,
        grid=(N // RB,),
        in_specs=[pl.BlockSpec((RB, D), lambda i: (i, 0)),
                  pl.BlockSpec((D, D), lambda i: (0, 0))],
        out_specs=pl.BlockSpec((RB, D), lambda i: (i, 0)),
        out_shape=jax.ShapeDtypeStruct((N, D), jnp.float32),
    )(x, w)


def _dinv_of(deg_ref):
    d0 = deg_ref[0][:, 0:1]
    d1 = deg_ref[1][:, 0:1]
    return lax.rsqrt(d0 + d1 + 1.0)


def _scale_body(deg_ref, h_ref, g_ref):
    g_ref[...] = h_ref[...] * _dinv_of(deg_ref)


def _scale(deg_parts, h):
    return pl.pallas_call(
        _scale_body,
        grid=(N // RB,),
        in_specs=[pl.BlockSpec((NC, RB, DEGW), lambda i: (0, i, 0)),
                  pl.BlockSpec((RB, D), lambda i: (i, 0))],
        out_specs=pl.BlockSpec((RB, D), lambda i: (i, 0)),
        out_shape=jax.ShapeDtypeStruct((N, D), jnp.float32),
    )(deg_parts, h)


def _combine_mm_body(s_ref, g_ref, deg_ref, b_ref, w_ref, o_ref):
    dinv = _dinv_of(deg_ref)
    pre = (s_ref[0] + s_ref[1] + g_ref[...]) * dinv + b_ref[...]
    act = jnp.maximum(pre, 0.0)
    o_ref[...] = jnp.dot(act, w_ref[...],
                         preferred_element_type=jnp.float32) * dinv


def _combine_mm(s_parts, g, deg_parts, b_row, w):
    return pl.pallas_call(
        _combine_mm_body,
        grid=(N // RB,),
        in_specs=[pl.BlockSpec((NC, RB, D), lambda i: (0, i, 0)),
                  pl.BlockSpec((RB, D), lambda i: (i, 0)),
                  pl.BlockSpec((NC, RB, DEGW), lambda i: (0, i, 0)),
                  pl.BlockSpec((1, D), lambda i: (0, 0)),
                  pl.BlockSpec((D, D), lambda i: (0, 0))],
        out_specs=pl.BlockSpec((RB, D), lambda i: (i, 0)),
        out_shape=jax.ShapeDtypeStruct((N, D), jnp.float32),
    )(s_parts, g, deg_parts, b_row, w)


def _combine_body(s_ref, g_ref, deg_ref, b_ref, o_ref):
    dinv = _dinv_of(deg_ref)
    pre = (s_ref[0] + s_ref[1] + g_ref[...]) * dinv + b_ref[...]
    o_ref[...] = jnp.maximum(pre, 0.0)


def _combine(s_parts, g, deg_parts, b_row):
    return pl.pallas_call(
        _combine_body,
        grid=(N // RB,),
        in_specs=[pl.BlockSpec((NC, RB, D), lambda i: (0, i, 0)),
                  pl.BlockSpec((RB, D), lambda i: (i, 0)),
                  pl.BlockSpec((NC, RB, DEGW), lambda i: (0, i, 0)),
                  pl.BlockSpec((1, D), lambda i: (0, 0))],
        out_specs=pl.BlockSpec((RB, D), lambda i: (i, 0)),
        out_shape=jax.ShapeDtypeStruct((N, D), jnp.float32),
    )(s_parts, g, deg_parts, b_row)


def kernel(x, edge_index, W1, b1, W2, b2):
    src = edge_index[0].reshape(NW, C, K)
    dst = edge_index[1].reshape(NW, C, K)
    ones_deg = jnp.ones((K, DEGW), jnp.float32)
    zeros_deg = jnp.zeros((N_PAD, DEGW), jnp.float32)
    zeros_rows = jnp.zeros((N_PAD, D), jnp.float32)

    deg_parts = _deg_call(dst, ones_deg, zeros_deg)
    h1 = _matmul(x, W1)
    g1 = _scale(deg_parts, h1)
    s_parts = _scat_call(g1, src, dst, zeros_rows)
    g2 = _combine_mm(s_parts, g1, deg_parts, b1.reshape(1, D), W2)
    t_parts = _scat_call(g2, src, dst, zeros_rows)
    return _combine(t_parts, g2, deg_parts, b2.reshape(1, D))


# speedup vs baseline: 21.9567x; 1.2598x over previous
import functools

import jax
import jax.numpy as jnp
from jax import lax
from jax.experimental import pallas as pl
from jax.experimental.pallas import tpu as pltpu
from jax.experimental.pallas import tpu_sc as plsc

N = 10000
D = 128
E = 320000

NC = 2
NS = 16
NW = NC * NS
EPW = E // NW
K = 100
C = EPW // K
N_PAD = 10240
RPT = N_PAD // NS
DEGW = 16

RB = 2000

_mesh = plsc.VectorSubcoreMesh(
    core_axis_name="c", subcore_axis_name="s", num_cores=NC, num_subcores=NS)


def _deg_body(dst_hbm, ones_hbm, zeros_hbm, deg_out, dst_v, ones_v, deg_sh):
    cid = lax.axis_index("c")
    sid = lax.axis_index("s")
    wid = sid * NC + cid
    pltpu.sync_copy(zeros_hbm.at[pl.ds(sid * RPT, RPT)],
                    deg_sh.at[pl.ds(sid * RPT, RPT)])
    pltpu.sync_copy(dst_hbm.at[wid], dst_v)
    pltpu.sync_copy(ones_hbm, ones_v)
    plsc.subcore_barrier()

    def chunk(j, carry):
        pltpu.sync_copy(ones_v, deg_sh.at[dst_v.at[j]], add=True)
        return carry

    lax.fori_loop(0, C, chunk, 0)
    plsc.subcore_barrier()
    pltpu.sync_copy(deg_sh.at[pl.ds(sid * RPT, RPT)],
                    deg_out.at[cid, pl.ds(sid * RPT, RPT)])


_deg_call = pl.kernel(
    _deg_body,
    out_type=jax.ShapeDtypeStruct((NC, N_PAD, DEGW), jnp.float32),
    mesh=_mesh,
    scratch_types=[
        pltpu.VMEM((C, K), jnp.int32),
        pltpu.VMEM((K, DEGW), jnp.float32),
        pltpu.VMEM_SHARED((N_PAD, DEGW), jnp.float32),
    ],
    compiler_params=pltpu.CompilerParams(use_tc_tiling_on_sc=False),
)


def _scat_body(g_hbm, src_hbm, dst_hbm, zeros_hbm, s_out,
               src_v, dst_v, rows0, rows1, s_sh, sem0, sem1):
    cid = lax.axis_index("c")
    sid = lax.axis_index("s")
    wid = sid * NC + cid
    pltpu.sync_copy(zeros_hbm.at[pl.ds(sid * RPT, RPT)],
                    s_sh.at[pl.ds(sid * RPT, RPT)])
    pltpu.sync_copy(src_hbm.at[wid], src_v)
    pltpu.sync_copy(dst_hbm.at[wid], dst_v)
    plsc.subcore_barrier()

    pltpu.async_copy(g_hbm.at[src_v.at[0]], rows0, sem0)

    def pair(i, carry):
        j0 = 2 * i
        j1 = j0 + 1
        pltpu.make_async_copy(g_hbm.at[src_v.at[j0]], rows0, sem0).wait()
        pltpu.async_copy(g_hbm.at[src_v.at[j1]], rows1, sem1)
        pltpu.sync_copy(rows0, s_sh.at[dst_v.at[j0]], add=True)
        pltpu.make_async_copy(g_hbm.at[src_v.at[j1]], rows1, sem1).wait()

        @pl.when(i + 1 < C // 2)
        def _():
            pltpu.async_copy(g_hbm.at[src_v.at[j0 + 2]], rows0, sem0)

        pltpu.sync_copy(rows1, s_sh.at[dst_v.at[j1]], add=True)
        return carry

    lax.fori_loop(0, C // 2, pair, 0)
    plsc.subcore_barrier()
    pltpu.sync_copy(s_sh.at[pl.ds(sid * RPT, RPT)],
                    s_out.at[cid, pl.ds(sid * RPT, RPT)])


_scat_call = pl.kernel(
    _scat_body,
    out_type=jax.ShapeDtypeStruct((NC, N_PAD, D), jnp.float32),
    mesh=_mesh,
    scratch_types=[
        pltpu.VMEM((C, K), jnp.int32),
        pltpu.VMEM((C, K), jnp.int32),
        pltpu.VMEM((K, D), jnp.float32),
        pltpu.VMEM((K, D), jnp.float32),
        pltpu.VMEM_SHARED((N_PAD, D), jnp.float32),
        pltpu.SemaphoreType.DMA,
        pltpu.SemaphoreType.DMA,
    ],
    compiler_params=pltpu.CompilerParams(use_tc_tiling_on_sc=False),
)


def _mm_body(x_ref, w_ref, o_ref):
    o_ref[...] = jnp.dot(x_ref[...], w_ref[...],
                         preferred_element_type=jnp.float32)


def _matmul(x, w):
    return pl.pallas_call(
        _mm_body,
        grid=(N // RB,),
        in_specs=[pl.BlockSpec((RB, D), lambda i: (i, 0)),
                  pl.BlockSpec((D, D), lambda i: (0, 0))],
        out_specs=pl.BlockSpec((RB, D), lambda i: (i, 0)),
        out_shape=jax.ShapeDtypeStruct((N, D), jnp.float32),
    )(x, w)


def _dinv_of(deg_ref):
    d0 = deg_ref[0][:, 0:1]
    d1 = deg_ref[1][:, 0:1]
    return lax.rsqrt(d0 + d1 + 1.0)


def _scale_body(deg_ref, h_ref, g_ref):
    g_ref[...] = h_ref[...] * _dinv_of(deg_ref)


def _scale(deg_parts, h):
    return pl.pallas_call(
        _scale_body,
        grid=(N // RB,),
        in_specs=[pl.BlockSpec((NC, RB, DEGW), lambda i: (0, i, 0)),
                  pl.BlockSpec((RB, D), lambda i: (i, 0))],
        out_specs=pl.BlockSpec((RB, D), lambda i: (i, 0)),
        out_shape=jax.ShapeDtypeStruct((N, D), jnp.float32),
    )(deg_parts, h)


def _combine_mm_body(s_ref, g_ref, deg_ref, b_ref, w_ref, o_ref):
    dinv = _dinv_of(deg_ref)
    pre = (s_ref[0] + s_ref[1] + g_ref[...]) * dinv + b_ref[...]
    act = jnp.maximum(pre, 0.0)
    o_ref[...] = jnp.dot(act, w_ref[...],
                         preferred_element_type=jnp.float32) * dinv


def _combine_mm(s_parts, g, deg_parts, b_row, w):
    return pl.pallas_call(
        _combine_mm_body,
        grid=(N // RB,),
        in_specs=[pl.BlockSpec((NC, RB, D), lambda i: (0, i, 0)),
                  pl.BlockSpec((RB, D), lambda i: (i, 0)),
                  pl.BlockSpec((NC, RB, DEGW), lambda i: (0, i, 0)),
                  pl.BlockSpec((1, D), lambda i: (0, 0)),
                  pl.BlockSpec((D, D), lambda i: (0, 0))],
        out_specs=pl.BlockSpec((RB, D), lambda i: (i, 0)),
        out_shape=jax.ShapeDtypeStruct((N, D), jnp.float32),
    )(s_parts, g, deg_parts, b_row, w)


def _combine_body(s_ref, g_ref, deg_ref, b_ref, o_ref):
    dinv = _dinv_of(deg_ref)
    pre = (s_ref[0] + s_ref[1] + g_ref[...]) * dinv + b_ref[...]
    o_ref[...] = jnp.maximum(pre, 0.0)


def _combine(s_parts, g, deg_parts, b_row):
    return pl.pallas_call(
        _combine_body,
        grid=(N // RB,),
        in_specs=[pl.BlockSpec((NC, RB, D), lambda i: (0, i, 0)),
                  pl.BlockSpec((RB, D), lambda i: (i, 0)),
                  pl.BlockSpec((NC, RB, DEGW), lambda i: (0, i, 0)),
                  pl.BlockSpec((1, D), lambda i: (0, 0))],
        out_specs=pl.BlockSpec((RB, D), lambda i: (i, 0)),
        out_shape=jax.ShapeDtypeStruct((N, D), jnp.float32),
    )(s_parts, g, deg_parts, b_row)


def kernel(x, edge_index, W1, b1, W2, b2):
    src = edge_index[0].reshape(NW, C, K)
    dst = edge_index[1].reshape(NW, C, K)
    ones_deg = jnp.ones((K, DEGW), jnp.float32)
    zeros_deg = jnp.zeros((N_PAD, DEGW), jnp.float32)
    zeros_rows = jnp.zeros((N_PAD, D), jnp.float32)

    deg_parts = _deg_call(dst, ones_deg, zeros_deg)
    h1 = _matmul(x, W1)
    g1 = _scale(deg_parts, h1)
    s_parts = _scat_call(g1, src, dst, zeros_rows)
    g2 = _combine_mm(s_parts, g1, deg_parts, b1.reshape(1, D), W2)
    t_parts = _scat_call(g2, src, dst, zeros_rows)
    return _combine(t_parts, g2, deg_parts, b2.reshape(1, D))


# speedup vs baseline: 22.4981x; 1.0247x over previous
import functools

import jax
import jax.numpy as jnp
from jax import lax
from jax.experimental import pallas as pl
from jax.experimental.pallas import tpu as pltpu
from jax.experimental.pallas import tpu_sc as plsc

N = 10000
D = 128
E = 320000

NC = 2
NS = 16
NW = NC * NS
EPW = E // NW
K = 100
C = EPW // K
N_PAD = 10240
RPT = N_PAD // NS
DEGW = 16

RB = 2000

_mesh = plsc.VectorSubcoreMesh(
    core_axis_name="c", subcore_axis_name="s", num_cores=NC, num_subcores=NS)


def _deg_body(dst_hbm, ones_hbm, zeros_hbm, deg_out, dst_v, ones_v, deg_sh,
              sem):
    cid = lax.axis_index("c")
    sid = lax.axis_index("s")
    wid = sid * NC + cid
    pltpu.sync_copy(zeros_hbm.at[pl.ds(sid * RPT, RPT)],
                    deg_sh.at[pl.ds(sid * RPT, RPT)])
    pltpu.sync_copy(dst_hbm.at[wid], dst_v)
    pltpu.sync_copy(ones_hbm, ones_v)
    plsc.subcore_barrier()

    def chunk(j, carry):
        pltpu.async_copy(ones_v, deg_sh.at[dst_v.at[j]], sem, add=True)
        return carry

    lax.fori_loop(0, C, chunk, 0)

    def drain(j, carry):
        pltpu.make_async_copy(ones_v, deg_sh.at[dst_v.at[0]], sem).wait()
        return carry

    lax.fori_loop(0, C, drain, 0)
    plsc.subcore_barrier()
    pltpu.sync_copy(deg_sh.at[pl.ds(sid * RPT, RPT)],
                    deg_out.at[cid, pl.ds(sid * RPT, RPT)])


_deg_call = pl.kernel(
    _deg_body,
    out_type=jax.ShapeDtypeStruct((NC, N_PAD, DEGW), jnp.float32),
    mesh=_mesh,
    scratch_types=[
        pltpu.VMEM((C, K), jnp.int32),
        pltpu.VMEM((K, DEGW), jnp.float32),
        pltpu.VMEM_SHARED((N_PAD, DEGW), jnp.float32),
        pltpu.SemaphoreType.DMA,
    ],
    compiler_params=pltpu.CompilerParams(use_tc_tiling_on_sc=False),
)


def _scat_body(g_hbm, src_hbm, dst_hbm, zeros_hbm, s_out,
               src_v, dst_v, rows0, rows1, s_sh, sem0, sem1):
    cid = lax.axis_index("c")
    sid = lax.axis_index("s")
    wid = sid * NC + cid
    pltpu.async_copy(zeros_hbm.at[pl.ds(sid * RPT, RPT)],
                     s_sh.at[pl.ds(sid * RPT, RPT)], sem0)
    pltpu.async_copy(src_hbm.at[wid], src_v, sem1)
    pltpu.async_copy(dst_hbm.at[wid], dst_v, sem1)
    pltpu.make_async_copy(zeros_hbm.at[pl.ds(sid * RPT, RPT)],
                          s_sh.at[pl.ds(sid * RPT, RPT)], sem0).wait()
    pltpu.make_async_copy(src_hbm.at[wid], src_v, sem1).wait()
    pltpu.make_async_copy(dst_hbm.at[wid], dst_v, sem1).wait()
    plsc.subcore_barrier()

    pltpu.async_copy(g_hbm.at[src_v.at[0]], rows0, sem0)

    def pair(i, carry):
        j0 = 2 * i
        j1 = j0 + 1
        pltpu.make_async_copy(g_hbm.at[src_v.at[j0]], rows0, sem0).wait()
        pltpu.async_copy(g_hbm.at[src_v.at[j1]], rows1, sem1)
        pltpu.sync_copy(rows0, s_sh.at[dst_v.at[j0]], add=True)
        pltpu.make_async_copy(g_hbm.at[src_v.at[j1]], rows1, sem1).wait()

        @pl.when(i + 1 < C // 2)
        def _():
            pltpu.async_copy(g_hbm.at[src_v.at[j0 + 2]], rows0, sem0)

        pltpu.sync_copy(rows1, s_sh.at[dst_v.at[j1]], add=True)
        return carry

    lax.fori_loop(0, C // 2, pair, 0)
    plsc.subcore_barrier()
    pltpu.sync_copy(s_sh.at[pl.ds(sid * RPT, RPT)],
                    s_out.at[cid, pl.ds(sid * RPT, RPT)])


_scat_call = pl.kernel(
    _scat_body,
    out_type=jax.ShapeDtypeStruct((NC, N_PAD, D), jnp.float32),
    mesh=_mesh,
    scratch_types=[
        pltpu.VMEM((C, K), jnp.int32),
        pltpu.VMEM((C, K), jnp.int32),
        pltpu.VMEM((K, D), jnp.float32),
        pltpu.VMEM((K, D), jnp.float32),
        pltpu.VMEM_SHARED((N_PAD, D), jnp.float32),
        pltpu.SemaphoreType.DMA,
        pltpu.SemaphoreType.DMA,
    ],
    compiler_params=pltpu.CompilerParams(use_tc_tiling_on_sc=False),
)


def _mm_body(x_ref, w_ref, o_ref):
    o_ref[...] = jnp.dot(x_ref[...], w_ref[...],
                         preferred_element_type=jnp.float32)


def _matmul(x, w):
    return pl.pallas_call(
        _mm_body,
        grid=(N // RB,),
        in_specs=[pl.BlockSpec((RB, D), lambda i: (i, 0)),
                  pl.BlockSpec((D, D), lambda i: (0, 0))],
        out_specs=pl.BlockSpec((RB, D), lambda i: (i, 0)),
        out_shape=jax.ShapeDtypeStruct((N, D), jnp.float32),
    )(x, w)


def _dinv_of(deg_ref):
    d0 = deg_ref[0][:, 0:1]
    d1 = deg_ref[1][:, 0:1]
    return lax.rsqrt(d0 + d1 + 1.0)


def _mm_scale_body(deg_ref, x_ref, w_ref, g_ref):
    h = jnp.dot(x_ref[...], w_ref[...], preferred_element_type=jnp.float32)
    g_ref[...] = h * _dinv_of(deg_ref)


def _mm_scale(deg_parts, x, w):
    return pl.pallas_call(
        _mm_scale_body,
        grid=(N // RB,),
        in_specs=[pl.BlockSpec((NC, RB, DEGW), lambda i: (0, i, 0)),
                  pl.BlockSpec((RB, D), lambda i: (i, 0)),
                  pl.BlockSpec((D, D), lambda i: (0, 0))],
        out_specs=pl.BlockSpec((RB, D), lambda i: (i, 0)),
        out_shape=jax.ShapeDtypeStruct((N, D), jnp.float32),
    )(deg_parts, x, w)


def _combine_mm_body(s_ref, g_ref, deg_ref, b_ref, w_ref, o_ref):
    dinv = _dinv_of(deg_ref)
    pre = (s_ref[0] + s_ref[1] + g_ref[...]) * dinv + b_ref[...]
    act = jnp.maximum(pre, 0.0)
    o_ref[...] = jnp.dot(act, w_ref[...],
                         preferred_element_type=jnp.float32) * dinv


def _combine_mm(s_parts, g, deg_parts, b_row, w):
    return pl.pallas_call(
        _combine_mm_body,
        grid=(N // RB,),
        in_specs=[pl.BlockSpec((NC, RB, D), lambda i: (0, i, 0)),
                  pl.BlockSpec((RB, D), lambda i: (i, 0)),
                  pl.BlockSpec((NC, RB, DEGW), lambda i: (0, i, 0)),
                  pl.BlockSpec((1, D), lambda i: (0, 0)),
                  pl.BlockSpec((D, D), lambda i: (0, 0))],
        out_specs=pl.BlockSpec((RB, D), lambda i: (i, 0)),
        out_shape=jax.ShapeDtypeStruct((N, D), jnp.float32),
    )(s_parts, g, deg_parts, b_row, w)


def _combine_body(s_ref, g_ref, deg_ref, b_ref, o_ref):
    dinv = _dinv_of(deg_ref)
    pre = (s_ref[0] + s_ref[1] + g_ref[...]) * dinv + b_ref[...]
    o_ref[...] = jnp.maximum(pre, 0.0)


def _combine(s_parts, g, deg_parts, b_row):
    return pl.pallas_call(
        _combine_body,
        grid=(N // RB,),
        in_specs=[pl.BlockSpec((NC, RB, D), lambda i: (0, i, 0)),
                  pl.BlockSpec((RB, D), lambda i: (i, 0)),
                  pl.BlockSpec((NC, RB, DEGW), lambda i: (0, i, 0)),
                  pl.BlockSpec((1, D), lambda i: (0, 0))],
        out_specs=pl.BlockSpec((RB, D), lambda i: (i, 0)),
        out_shape=jax.ShapeDtypeStruct((N, D), jnp.float32),
    )(s_parts, g, deg_parts, b_row)


def kernel(x, edge_index, W1, b1, W2, b2):
    src = edge_index[0].reshape(NW, C, K)
    dst = edge_index[1].reshape(NW, C, K)
    ones_deg = jnp.ones((K, DEGW), jnp.float32)
    zeros_deg = jnp.zeros((N_PAD, DEGW), jnp.float32)
    zeros_rows = jnp.zeros((N_PAD, D), jnp.float32)

    deg_parts = _deg_call(dst, ones_deg, zeros_deg)
    g1 = _mm_scale(deg_parts, x, W1)
    s_parts = _scat_call(g1, src, dst, zeros_rows)
    g2 = _combine_mm(s_parts, g1, deg_parts, b1.reshape(1, D), W2)
    t_parts = _scat_call(g2, src, dst, zeros_rows)
    return _combine(t_parts, g2, deg_parts, b2.reshape(1, D))


# speedup vs baseline: 26.4781x; 1.1769x over previous
import functools

import jax
import jax.numpy as jnp
from jax import lax
from jax.experimental import pallas as pl
from jax.experimental.pallas import tpu as pltpu
from jax.experimental.pallas import tpu_sc as plsc

N = 10000
D = 128
E = 320000

NC = 2
NS = 16
NW = NC * NS
EPW = E // NW
K = 40
C = EPW // K
NB = 5
N_PAD = 10240
RPT = N_PAD // NS
DEGW = 16

RB = 2000

_mesh = plsc.VectorSubcoreMesh(
    core_axis_name="c", subcore_axis_name="s", num_cores=NC, num_subcores=NS)


def _deg_body(dst_hbm, ones_hbm, zeros_hbm, deg_out, dst_v, ones_v, deg_sh,
              sem):
    cid = lax.axis_index("c")
    sid = lax.axis_index("s")
    wid = sid * NC + cid
    pltpu.sync_copy(zeros_hbm.at[pl.ds(sid * RPT, RPT)],
                    deg_sh.at[pl.ds(sid * RPT, RPT)])
    pltpu.sync_copy(dst_hbm.at[wid], dst_v)
    pltpu.sync_copy(ones_hbm, ones_v)
    plsc.subcore_barrier()

    def chunk(j, carry):
        pltpu.async_copy(ones_v, deg_sh.at[dst_v.at[j]], sem, add=True)
        return carry

    lax.fori_loop(0, C, chunk, 0)

    def drain(j, carry):
        pltpu.make_async_copy(ones_v, deg_sh.at[dst_v.at[0]], sem).wait()
        return carry

    lax.fori_loop(0, C, drain, 0)
    plsc.subcore_barrier()
    pltpu.sync_copy(deg_sh.at[pl.ds(sid * RPT, RPT)],
                    deg_out.at[cid, pl.ds(sid * RPT, RPT)])


_deg_call = pl.kernel(
    _deg_body,
    out_type=jax.ShapeDtypeStruct((NC, N_PAD, DEGW), jnp.float32),
    mesh=_mesh,
    scratch_types=[
        pltpu.VMEM((C, K), jnp.int32),
        pltpu.VMEM((K, DEGW), jnp.float32),
        pltpu.VMEM_SHARED((N_PAD, DEGW), jnp.float32),
        pltpu.SemaphoreType.DMA,
    ],
    compiler_params=pltpu.CompilerParams(use_tc_tiling_on_sc=False),
)


def _scat_body(g_hbm, src_hbm, dst_hbm, zeros_hbm, s_out,
               src_v, dst_v, rows, s_sh, gsems, ssems):
    cid = lax.axis_index("c")
    sid = lax.axis_index("s")
    wid = sid * NC + cid
    pltpu.async_copy(zeros_hbm.at[pl.ds(sid * RPT, RPT)],
                     s_sh.at[pl.ds(sid * RPT, RPT)], gsems.at[0])
    pltpu.async_copy(src_hbm.at[wid], src_v, gsems.at[1])
    pltpu.async_copy(dst_hbm.at[wid], dst_v, gsems.at[1])
    pltpu.make_async_copy(zeros_hbm.at[pl.ds(sid * RPT, RPT)],
                          s_sh.at[pl.ds(sid * RPT, RPT)], gsems.at[0]).wait()
    pltpu.make_async_copy(src_hbm.at[wid], src_v, gsems.at[1]).wait()
    pltpu.make_async_copy(dst_hbm.at[wid], dst_v, gsems.at[1]).wait()
    plsc.subcore_barrier()

    for b in range(NB):
        pltpu.async_copy(g_hbm.at[src_v.at[b]], rows.at[b], gsems.at[b])

    def group(g, carry):
        base = g * NB
        for b in range(NB):
            j = base + b
            pltpu.make_async_copy(g_hbm.at[src_v.at[j]], rows.at[b],
                                  gsems.at[b]).wait()
            pltpu.async_copy(rows.at[b], s_sh.at[dst_v.at[j]], ssems.at[b],
                             add=True)
        for b in range(NB):
            j2 = base + NB + b

            @pl.when(j2 < C)
            def _(b=b, j2=j2):
                pltpu.make_async_copy(rows.at[b], s_sh.at[dst_v.at[0]],
                                      ssems.at[b]).wait()
                pltpu.async_copy(g_hbm.at[src_v.at[j2]], rows.at[b],
                                 gsems.at[b])
        return carry

    lax.fori_loop(0, C // NB, group, 0)
    for b in range(NB):
        pltpu.make_async_copy(rows.at[b], s_sh.at[dst_v.at[0]],
                              ssems.at[b]).wait()
    plsc.subcore_barrier()
    pltpu.sync_copy(s_sh.at[pl.ds(sid * RPT, RPT)],
                    s_out.at[cid, pl.ds(sid * RPT, RPT)])


_scat_call = pl.kernel(
    _scat_body,
    out_type=jax.ShapeDtypeStruct((NC, N_PAD, D), jnp.float32),
    mesh=_mesh,
    scratch_types=[
        pltpu.VMEM((C, K), jnp.int32),
        pltpu.VMEM((C, K), jnp.int32),
        pltpu.VMEM((NB, K, D), jnp.float32),
        pltpu.VMEM_SHARED((N_PAD, D), jnp.float32),
        pltpu.SemaphoreType.DMA((NB,)),
        pltpu.SemaphoreType.DMA((NB,)),
    ],
    compiler_params=pltpu.CompilerParams(use_tc_tiling_on_sc=False),
)


def _mm_body(x_ref, w_ref, o_ref):
    o_ref[...] = jnp.dot(x_ref[...], w_ref[...],
                         preferred_element_type=jnp.float32)


def _matmul(x, w):
    return pl.pallas_call(
        _mm_body,
        grid=(N // RB,),
        in_specs=[pl.BlockSpec((RB, D), lambda i: (i, 0)),
                  pl.BlockSpec((D, D), lambda i: (0, 0))],
        out_specs=pl.BlockSpec((RB, D), lambda i: (i, 0)),
        out_shape=jax.ShapeDtypeStruct((N, D), jnp.float32),
    )(x, w)


def _dinv_of(deg_ref):
    d0 = deg_ref[0][:, 0:1]
    d1 = deg_ref[1][:, 0:1]
    return lax.rsqrt(d0 + d1 + 1.0)


def _mm_scale_body(deg_ref, x_ref, w_ref, g_ref):
    h = jnp.dot(x_ref[...], w_ref[...], preferred_element_type=jnp.float32)
    g_ref[...] = h * _dinv_of(deg_ref)


def _mm_scale(deg_parts, x, w):
    return pl.pallas_call(
        _mm_scale_body,
        grid=(N // RB,),
        in_specs=[pl.BlockSpec((NC, RB, DEGW), lambda i: (0, i, 0)),
                  pl.BlockSpec((RB, D), lambda i: (i, 0)),
                  pl.BlockSpec((D, D), lambda i: (0, 0))],
        out_specs=pl.BlockSpec((RB, D), lambda i: (i, 0)),
        out_shape=jax.ShapeDtypeStruct((N, D), jnp.float32),
    )(deg_parts, x, w)


def _combine_mm_body(s_ref, g_ref, deg_ref, b_ref, w_ref, o_ref):
    dinv = _dinv_of(deg_ref)
    pre = (s_ref[0] + s_ref[1] + g_ref[...]) * dinv + b_ref[...]
    act = jnp.maximum(pre, 0.0)
    o_ref[...] = jnp.dot(act, w_ref[...],
                         preferred_element_type=jnp.float32) * dinv


def _combine_mm(s_parts, g, deg_parts, b_row, w):
    return pl.pallas_call(
        _combine_mm_body,
        grid=(N // RB,),
        in_specs=[pl.BlockSpec((NC, RB, D), lambda i: (0, i, 0)),
                  pl.BlockSpec((RB, D), lambda i: (i, 0)),
                  pl.BlockSpec((NC, RB, DEGW), lambda i: (0, i, 0)),
                  pl.BlockSpec((1, D), lambda i: (0, 0)),
                  pl.BlockSpec((D, D), lambda i: (0, 0))],
        out_specs=pl.BlockSpec((RB, D), lambda i: (i, 0)),
        out_shape=jax.ShapeDtypeStruct((N, D), jnp.float32),
    )(s_parts, g, deg_parts, b_row, w)


def _combine_body(s_ref, g_ref, deg_ref, b_ref, o_ref):
    dinv = _dinv_of(deg_ref)
    pre = (s_ref[0] + s_ref[1] + g_ref[...]) * dinv + b_ref[...]
    o_ref[...] = jnp.maximum(pre, 0.0)


def _combine(s_parts, g, deg_parts, b_row):
    return pl.pallas_call(
        _combine_body,
        grid=(N // RB,),
        in_specs=[pl.BlockSpec((NC, RB, D), lambda i: (0, i, 0)),
                  pl.BlockSpec((RB, D), lambda i: (i, 0)),
                  pl.BlockSpec((NC, RB, DEGW), lambda i: (0, i, 0)),
                  pl.BlockSpec((1, D), lambda i: (0, 0))],
        out_specs=pl.BlockSpec((RB, D), lambda i: (i, 0)),
        out_shape=jax.ShapeDtypeStruct((N, D), jnp.float32),
    )(s_parts, g, deg_parts, b_row)


def kernel(x, edge_index, W1, b1, W2, b2):
    src = edge_index[0].reshape(NW, C, K)
    dst = edge_index[1].reshape(NW, C, K)
    ones_deg = jnp.ones((K, DEGW), jnp.float32)
    zeros_deg = jnp.zeros((N_PAD, DEGW), jnp.float32)
    zeros_rows = jnp.zeros((N_PAD, D), jnp.float32)

    deg_parts = _deg_call(dst, ones_deg, zeros_deg)
    g1 = _mm_scale(deg_parts, x, W1)
    s_parts = _scat_call(g1, src, dst, zeros_rows)
    g2 = _combine_mm(s_parts, g1, deg_parts, b1.reshape(1, D), W2)
    t_parts = _scat_call(g2, src, dst, zeros_rows)
    return _combine(t_parts, g2, deg_parts, b2.reshape(1, D))


# speedup vs baseline: 28.9435x; 1.0931x over previous
import functools

import jax
import jax.numpy as jnp
from jax import lax
from jax.experimental import pallas as pl
from jax.experimental.pallas import tpu as pltpu
from jax.experimental.pallas import tpu_sc as plsc

N = 10000
D = 128
E = 320000

NC = 2
NS = 16
NW = NC * NS
EPW = E // NW
K = 40
C = EPW // K
NB = 5
N_PAD = 10240
RPT = N_PAD // NS
DEGW = 16

RB = 2000

_mesh = plsc.VectorSubcoreMesh(
    core_axis_name="c", subcore_axis_name="s", num_cores=NC, num_subcores=NS)


def _deg_body(dst_hbm, ones_hbm, zeros_hbm, deg_out, dst_v, ones_v, deg_sh,
              sem):
    cid = lax.axis_index("c")
    sid = lax.axis_index("s")
    wid = sid * NC + cid
    pltpu.sync_copy(zeros_hbm.at[pl.ds(sid * RPT, RPT)],
                    deg_sh.at[pl.ds(sid * RPT, RPT)])
    pltpu.sync_copy(dst_hbm.at[wid], dst_v)
    pltpu.sync_copy(ones_hbm, ones_v)
    plsc.subcore_barrier()

    def chunk(j, carry):
        pltpu.async_copy(ones_v, deg_sh.at[dst_v.at[j]], sem, add=True)
        return carry

    lax.fori_loop(0, C, chunk, 0)

    def drain(j, carry):
        pltpu.make_async_copy(ones_v, deg_sh.at[dst_v.at[0]], sem).wait()
        return carry

    lax.fori_loop(0, C, drain, 0)
    plsc.subcore_barrier()
    pltpu.sync_copy(deg_sh.at[pl.ds(sid * RPT, RPT)],
                    deg_out.at[cid, pl.ds(sid * RPT, RPT)])


_deg_call = pl.kernel(
    _deg_body,
    out_type=jax.ShapeDtypeStruct((NC, N_PAD, DEGW), jnp.float32),
    mesh=_mesh,
    scratch_types=[
        pltpu.VMEM((C, K), jnp.int32),
        pltpu.VMEM((K, DEGW), jnp.float32),
        pltpu.VMEM_SHARED((N_PAD, DEGW), jnp.float32),
        pltpu.SemaphoreType.DMA,
    ],
    compiler_params=pltpu.CompilerParams(use_tc_tiling_on_sc=False),
)


def _scat_body(g_hbm, src_hbm, dst_hbm, zeros_hbm, s_out,
               src_v, dst_v, rows, s_sh, gsems, ssems):
    cid = lax.axis_index("c")
    sid = lax.axis_index("s")
    wid = sid * NC + cid
    pltpu.async_copy(zeros_hbm.at[pl.ds(sid * RPT, RPT)],
                     s_sh.at[pl.ds(sid * RPT, RPT)], gsems.at[0])
    pltpu.async_copy(src_hbm.at[wid], src_v, gsems.at[1])
    pltpu.async_copy(dst_hbm.at[wid], dst_v, gsems.at[1])
    pltpu.make_async_copy(zeros_hbm.at[pl.ds(sid * RPT, RPT)],
                          s_sh.at[pl.ds(sid * RPT, RPT)], gsems.at[0]).wait()
    pltpu.make_async_copy(src_hbm.at[wid], src_v, gsems.at[1]).wait()
    pltpu.make_async_copy(dst_hbm.at[wid], dst_v, gsems.at[1]).wait()
    plsc.subcore_barrier()

    for b in range(NB):
        pltpu.async_copy(g_hbm.at[src_v.at[b]], rows.at[b], gsems.at[b])

    def group(g, carry):
        base = g * NB
        for b in range(NB):
            j = base + b
            pltpu.make_async_copy(g_hbm.at[src_v.at[j]], rows.at[b],
                                  gsems.at[b]).wait()
            pltpu.async_copy(rows.at[b], s_sh.at[dst_v.at[j]], ssems.at[b],
                             add=True)
        for b in range(NB):
            j2 = base + NB + b

            @pl.when(j2 < C)
            def _(b=b, j2=j2):
                pltpu.make_async_copy(rows.at[b], s_sh.at[dst_v.at[0]],
                                      ssems.at[b]).wait()
                pltpu.async_copy(g_hbm.at[src_v.at[j2]], rows.at[b],
                                 gsems.at[b])
        return carry

    lax.fori_loop(0, C // NB, group, 0)
    for b in range(NB):
        pltpu.make_async_copy(rows.at[b], s_sh.at[dst_v.at[0]],
                              ssems.at[b]).wait()
    plsc.subcore_barrier()
    pltpu.sync_copy(s_sh.at[pl.ds(sid * RPT, RPT)],
                    s_out.at[cid, pl.ds(sid * RPT, RPT)])


_scat_call = pl.kernel(
    _scat_body,
    out_type=jax.ShapeDtypeStruct((NC, N_PAD, D), jnp.bfloat16),
    mesh=_mesh,
    scratch_types=[
        pltpu.VMEM((C, K), jnp.int32),
        pltpu.VMEM((C, K), jnp.int32),
        pltpu.VMEM((NB, K, D), jnp.bfloat16),
        pltpu.VMEM_SHARED((N_PAD, D), jnp.bfloat16),
        pltpu.SemaphoreType.DMA((NB,)),
        pltpu.SemaphoreType.DMA((NB,)),
    ],
    compiler_params=pltpu.CompilerParams(use_tc_tiling_on_sc=False),
)


def _mm_body(x_ref, w_ref, o_ref):
    o_ref[...] = jnp.dot(x_ref[...], w_ref[...],
                         preferred_element_type=jnp.float32)


def _matmul(x, w):
    return pl.pallas_call(
        _mm_body,
        grid=(N // RB,),
        in_specs=[pl.BlockSpec((RB, D), lambda i: (i, 0)),
                  pl.BlockSpec((D, D), lambda i: (0, 0))],
        out_specs=pl.BlockSpec((RB, D), lambda i: (i, 0)),
        out_shape=jax.ShapeDtypeStruct((N, D), jnp.float32),
    )(x, w)


def _dinv_of(deg_ref):
    d0 = deg_ref[0][:, 0:1]
    d1 = deg_ref[1][:, 0:1]
    return lax.rsqrt(d0 + d1 + 1.0)


def _mm_scale_body(deg_ref, x_ref, w_ref, g_ref, gb_ref):
    h = jnp.dot(x_ref[...], w_ref[...], preferred_element_type=jnp.float32)
    g = h * _dinv_of(deg_ref)
    g_ref[...] = g
    gb_ref[...] = g.astype(jnp.bfloat16)


def _mm_scale(deg_parts, x, w):
    return pl.pallas_call(
        _mm_scale_body,
        grid=(N // RB,),
        in_specs=[pl.BlockSpec((NC, RB, DEGW), lambda i: (0, i, 0)),
                  pl.BlockSpec((RB, D), lambda i: (i, 0)),
                  pl.BlockSpec((D, D), lambda i: (0, 0))],
        out_specs=[pl.BlockSpec((RB, D), lambda i: (i, 0)),
                   pl.BlockSpec((RB, D), lambda i: (i, 0))],
        out_shape=[jax.ShapeDtypeStruct((N, D), jnp.float32),
                   jax.ShapeDtypeStruct((N, D), jnp.bfloat16)],
    )(deg_parts, x, w)


def _combine_mm_body(s_ref, g_ref, deg_ref, b_ref, w_ref, o_ref, ob_ref):
    dinv = _dinv_of(deg_ref)
    s = s_ref[0].astype(jnp.float32) + s_ref[1].astype(jnp.float32)
    pre = (s + g_ref[...]) * dinv + b_ref[...]
    act = jnp.maximum(pre, 0.0)
    g2 = jnp.dot(act, w_ref[...],
                 preferred_element_type=jnp.float32) * dinv
    o_ref[...] = g2
    ob_ref[...] = g2.astype(jnp.bfloat16)


def _combine_mm(s_parts, g, deg_parts, b_row, w):
    return pl.pallas_call(
        _combine_mm_body,
        grid=(N // RB,),
        in_specs=[pl.BlockSpec((NC, RB, D), lambda i: (0, i, 0)),
                  pl.BlockSpec((RB, D), lambda i: (i, 0)),
                  pl.BlockSpec((NC, RB, DEGW), lambda i: (0, i, 0)),
                  pl.BlockSpec((1, D), lambda i: (0, 0)),
                  pl.BlockSpec((D, D), lambda i: (0, 0))],
        out_specs=[pl.BlockSpec((RB, D), lambda i: (i, 0)),
                   pl.BlockSpec((RB, D), lambda i: (i, 0))],
        out_shape=[jax.ShapeDtypeStruct((N, D), jnp.float32),
                   jax.ShapeDtypeStruct((N, D), jnp.bfloat16)],
    )(s_parts, g, deg_parts, b_row, w)


def _combine_body(s_ref, g_ref, deg_ref, b_ref, o_ref):
    dinv = _dinv_of(deg_ref)
    s = s_ref[0].astype(jnp.float32) + s_ref[1].astype(jnp.float32)
    pre = (s + g_ref[...]) * dinv + b_ref[...]
    o_ref[...] = jnp.maximum(pre, 0.0)


def _combine(s_parts, g, deg_parts, b_row):
    return pl.pallas_call(
        _combine_body,
        grid=(N // RB,),
        in_specs=[pl.BlockSpec((NC, RB, D), lambda i: (0, i, 0)),
                  pl.BlockSpec((RB, D), lambda i: (i, 0)),
                  pl.BlockSpec((NC, RB, DEGW), lambda i: (0, i, 0)),
                  pl.BlockSpec((1, D), lambda i: (0, 0))],
        out_specs=pl.BlockSpec((RB, D), lambda i: (i, 0)),
        out_shape=jax.ShapeDtypeStruct((N, D), jnp.float32),
    )(s_parts, g, deg_parts, b_row)


def kernel(x, edge_index, W1, b1, W2, b2):
    src = edge_index[0].reshape(NW, C, K)
    dst = edge_index[1].reshape(NW, C, K)
    ones_deg = jnp.ones((K, DEGW), jnp.float32)
    zeros_deg = jnp.zeros((N_PAD, DEGW), jnp.float32)
    zeros_rows = jnp.zeros((N_PAD, D), jnp.bfloat16)

    deg_parts = _deg_call(dst, ones_deg, zeros_deg)
    g1, g1b = _mm_scale(deg_parts, x, W1)
    s_parts = _scat_call(g1b, src, dst, zeros_rows)
    g2, g2b = _combine_mm(s_parts, g1, deg_parts, b1.reshape(1, D), W2)
    t_parts = _scat_call(g2b, src, dst, zeros_rows)
    return _combine(t_parts, g2, deg_parts, b2.reshape(1, D))


# speedup vs baseline: 30.8079x; 1.0644x over previous
import functools

import jax
import jax.numpy as jnp
from jax import lax
from jax.experimental import pallas as pl
from jax.experimental.pallas import tpu as pltpu
from jax.experimental.pallas import tpu_sc as plsc

N = 10000
D = 128
E = 320000

NC = 2
NS = 16
NW = NC * NS
EPW = E // NW
K = 100
C = EPW // K
NB = 5
N_PAD = 10240
RPT = N_PAD // NS
DEGW = 16

RB = 2000

_mesh = plsc.VectorSubcoreMesh(
    core_axis_name="c", subcore_axis_name="s", num_cores=NC, num_subcores=NS)


def _deg_body(dst_hbm, ones_hbm, zeros_hbm, deg_out, dst_v, ones_v, deg_sh,
              sem):
    cid = lax.axis_index("c")
    sid = lax.axis_index("s")
    wid = sid * NC + cid
    pltpu.sync_copy(zeros_hbm.at[pl.ds(sid * RPT, RPT)],
                    deg_sh.at[pl.ds(sid * RPT, RPT)])
    pltpu.sync_copy(dst_hbm.at[wid], dst_v)
    pltpu.sync_copy(ones_hbm, ones_v)
    plsc.subcore_barrier()

    def chunk(j, carry):
        pltpu.async_copy(ones_v, deg_sh.at[dst_v.at[j]], sem, add=True)
        return carry

    lax.fori_loop(0, C, chunk, 0)

    def drain(j, carry):
        pltpu.make_async_copy(ones_v, deg_sh.at[dst_v.at[0]], sem).wait()
        return carry

    lax.fori_loop(0, C, drain, 0)
    plsc.subcore_barrier()
    pltpu.sync_copy(deg_sh.at[pl.ds(sid * RPT, RPT)],
                    deg_out.at[cid, pl.ds(sid * RPT, RPT)])


_deg_call = pl.kernel(
    _deg_body,
    out_type=jax.ShapeDtypeStruct((NC, N_PAD, DEGW), jnp.float32),
    mesh=_mesh,
    scratch_types=[
        pltpu.VMEM((C, K), jnp.int32),
        pltpu.VMEM((K, DEGW), jnp.float32),
        pltpu.VMEM_SHARED((N_PAD, DEGW), jnp.float32),
        pltpu.SemaphoreType.DMA,
    ],
    compiler_params=pltpu.CompilerParams(use_tc_tiling_on_sc=False),
)


def _scat_body(g_hbm, src_hbm, dst_hbm, zeros_hbm, s_out,
               src_v, dst_v, rows, s_sh, gsems, ssems):
    cid = lax.axis_index("c")
    sid = lax.axis_index("s")
    wid = sid * NC + cid
    pltpu.async_copy(zeros_hbm.at[pl.ds(sid * RPT, RPT)],
                     s_sh.at[pl.ds(sid * RPT, RPT)], gsems.at[0])
    pltpu.async_copy(src_hbm.at[wid], src_v, gsems.at[1])
    pltpu.async_copy(dst_hbm.at[wid], dst_v, gsems.at[1])
    pltpu.make_async_copy(zeros_hbm.at[pl.ds(sid * RPT, RPT)],
                          s_sh.at[pl.ds(sid * RPT, RPT)], gsems.at[0]).wait()
    pltpu.make_async_copy(src_hbm.at[wid], src_v, gsems.at[1]).wait()
    pltpu.make_async_copy(dst_hbm.at[wid], dst_v, gsems.at[1]).wait()
    plsc.subcore_barrier()

    for b in range(NB):
        pltpu.async_copy(g_hbm.at[src_v.at[b]], rows.at[b], gsems.at[b])

    def group(g, carry):
        base = g * NB
        for b in range(NB):
            j = base + b
            pltpu.make_async_copy(g_hbm.at[src_v.at[j]], rows.at[b],
                                  gsems.at[b]).wait()
            pltpu.async_copy(rows.at[b], s_sh.at[dst_v.at[j]], ssems.at[b],
                             add=True)
        for b in range(NB):
            j2 = base + NB + b

            @pl.when(j2 < C)
            def _(b=b, j2=j2):
                pltpu.make_async_copy(rows.at[b], s_sh.at[dst_v.at[0]],
                                      ssems.at[b]).wait()
                pltpu.async_copy(g_hbm.at[src_v.at[j2]], rows.at[b],
                                 gsems.at[b])
        return carry

    lax.fori_loop(0, C // NB, group, 0)
    for b in range(NB):
        pltpu.make_async_copy(rows.at[b], s_sh.at[dst_v.at[0]],
                              ssems.at[b]).wait()
    plsc.subcore_barrier()
    pltpu.sync_copy(s_sh.at[pl.ds(sid * RPT, RPT)],
                    s_out.at[cid, pl.ds(sid * RPT, RPT)])


_scat_call = pl.kernel(
    _scat_body,
    out_type=jax.ShapeDtypeStruct((NC, N_PAD, D), jnp.bfloat16),
    mesh=_mesh,
    scratch_types=[
        pltpu.VMEM((C, K), jnp.int32),
        pltpu.VMEM((C, K), jnp.int32),
        pltpu.VMEM((NB, K, D), jnp.bfloat16),
        pltpu.VMEM_SHARED((N_PAD, D), jnp.bfloat16),
        pltpu.SemaphoreType.DMA((NB,)),
        pltpu.SemaphoreType.DMA((NB,)),
    ],
    compiler_params=pltpu.CompilerParams(use_tc_tiling_on_sc=False),
)


def _mm_body(x_ref, w_ref, o_ref):
    o_ref[...] = jnp.dot(x_ref[...], w_ref[...],
                         preferred_element_type=jnp.float32)


def _matmul(x, w):
    return pl.pallas_call(
        _mm_body,
        grid=(N // RB,),
        in_specs=[pl.BlockSpec((RB, D), lambda i: (i, 0)),
                  pl.BlockSpec((D, D), lambda i: (0, 0))],
        out_specs=pl.BlockSpec((RB, D), lambda i: (i, 0)),
        out_shape=jax.ShapeDtypeStruct((N, D), jnp.float32),
    )(x, w)


def _dinv_of(deg_ref):
    d0 = deg_ref[0][:, 0:1]
    d1 = deg_ref[1][:, 0:1]
    return lax.rsqrt(d0 + d1 + 1.0)


def _mm_scale_body(deg_ref, x_ref, w_ref, g_ref, gb_ref):
    h = jnp.dot(x_ref[...], w_ref[...], preferred_element_type=jnp.float32)
    g = h * _dinv_of(deg_ref)
    g_ref[...] = g
    gb_ref[...] = g.astype(jnp.bfloat16)


def _mm_scale(deg_parts, x, w):
    return pl.pallas_call(
        _mm_scale_body,
        grid=(N // RB,),
        in_specs=[pl.BlockSpec((NC, RB, DEGW), lambda i: (0, i, 0)),
                  pl.BlockSpec((RB, D), lambda i: (i, 0)),
                  pl.BlockSpec((D, D), lambda i: (0, 0))],
        out_specs=[pl.BlockSpec((RB, D), lambda i: (i, 0)),
                   pl.BlockSpec((RB, D), lambda i: (i, 0))],
        out_shape=[jax.ShapeDtypeStruct((N, D), jnp.float32),
                   jax.ShapeDtypeStruct((N, D), jnp.bfloat16)],
    )(deg_parts, x, w)


def _combine_mm_body(s_ref, g_ref, deg_ref, b_ref, w_ref, o_ref, ob_ref):
    dinv = _dinv_of(deg_ref)
    s = s_ref[0].astype(jnp.float32) + s_ref[1].astype(jnp.float32)
    pre = (s + g_ref[...]) * dinv + b_ref[...]
    act = jnp.maximum(pre, 0.0)
    g2 = jnp.dot(act, w_ref[...],
                 preferred_element_type=jnp.float32) * dinv
    o_ref[...] = g2
    ob_ref[...] = g2.astype(jnp.bfloat16)


def _combine_mm(s_parts, g, deg_parts, b_row, w):
    return pl.pallas_call(
        _combine_mm_body,
        grid=(N // RB,),
        in_specs=[pl.BlockSpec((NC, RB, D), lambda i: (0, i, 0)),
                  pl.BlockSpec((RB, D), lambda i: (i, 0)),
                  pl.BlockSpec((NC, RB, DEGW), lambda i: (0, i, 0)),
                  pl.BlockSpec((1, D), lambda i: (0, 0)),
                  pl.BlockSpec((D, D), lambda i: (0, 0))],
        out_specs=[pl.BlockSpec((RB, D), lambda i: (i, 0)),
                   pl.BlockSpec((RB, D), lambda i: (i, 0))],
        out_shape=[jax.ShapeDtypeStruct((N, D), jnp.float32),
                   jax.ShapeDtypeStruct((N, D), jnp.bfloat16)],
    )(s_parts, g, deg_parts, b_row, w)


def _combine_body(s_ref, g_ref, deg_ref, b_ref, o_ref):
    dinv = _dinv_of(deg_ref)
    s = s_ref[0].astype(jnp.float32) + s_ref[1].astype(jnp.float32)
    pre = (s + g_ref[...]) * dinv + b_ref[...]
    o_ref[...] = jnp.maximum(pre, 0.0)


def _combine(s_parts, g, deg_parts, b_row):
    return pl.pallas_call(
        _combine_body,
        grid=(N // RB,),
        in_specs=[pl.BlockSpec((NC, RB, D), lambda i: (0, i, 0)),
                  pl.BlockSpec((RB, D), lambda i: (i, 0)),
                  pl.BlockSpec((NC, RB, DEGW), lambda i: (0, i, 0)),
                  pl.BlockSpec((1, D), lambda i: (0, 0))],
        out_specs=pl.BlockSpec((RB, D), lambda i: (i, 0)),
        out_shape=jax.ShapeDtypeStruct((N, D), jnp.float32),
    )(s_parts, g, deg_parts, b_row)


def kernel(x, edge_index, W1, b1, W2, b2):
    src = edge_index[0].reshape(NW, C, K)
    dst = edge_index[1].reshape(NW, C, K)
    ones_deg = jnp.ones((K, DEGW), jnp.float32)
    zeros_deg = jnp.zeros((N_PAD, DEGW), jnp.float32)
    zeros_rows = jnp.zeros((N_PAD, D), jnp.bfloat16)

    deg_parts = _deg_call(dst, ones_deg, zeros_deg)
    g1, g1b = _mm_scale(deg_parts, x, W1)
    s_parts = _scat_call(g1b, src, dst, zeros_rows)
    g2, g2b = _combine_mm(s_parts, g1, deg_parts, b1.reshape(1, D), W2)
    t_parts = _scat_call(g2b, src, dst, zeros_rows)
    return _combine(t_parts, g2, deg_parts, b2.reshape(1, D))


# speedup vs baseline: 32.6249x; 1.0590x over previous
import functools

import jax
import jax.numpy as jnp
from jax import lax
from jax.experimental import pallas as pl
from jax.experimental.pallas import tpu as pltpu
from jax.experimental.pallas import tpu_sc as plsc

N = 10000
D = 128
E = 320000

NC = 2
NS = 16
NW = NC * NS
EPW = E // NW
K = 80
C = EPW // K
NB = 5
N_PAD = 10240
RPT = N_PAD // NS
DEGW = 16

RB = 2000

_mesh = plsc.VectorSubcoreMesh(
    core_axis_name="c", subcore_axis_name="s", num_cores=NC, num_subcores=NS)


def _deg_body(ei_hbm, ones_hbm, zeros_hbm, deg_out, dst_v, ones_v, deg_sh,
              sem):
    cid = lax.axis_index("c")
    sid = lax.axis_index("s")
    wid = sid * NC + cid
    pltpu.sync_copy(zeros_hbm.at[pl.ds(sid * RPT, RPT)],
                    deg_sh.at[pl.ds(sid * RPT, RPT)])
    pltpu.sync_copy(ei_hbm.at[pl.ds(E + wid * EPW, EPW)], dst_v)
    pltpu.sync_copy(ones_hbm, ones_v)
    plsc.subcore_barrier()

    def chunk(j, carry):
        pltpu.async_copy(ones_v, deg_sh.at[dst_v.at[pl.ds(j * K, K)]],
                         sem, add=True)
        return carry

    lax.fori_loop(0, C, chunk, 0)

    def drain(j, carry):
        pltpu.make_async_copy(ones_v, deg_sh.at[dst_v.at[pl.ds(0, K)]],
                              sem).wait()
        return carry

    lax.fori_loop(0, C, drain, 0)
    plsc.subcore_barrier()
    pltpu.sync_copy(deg_sh.at[pl.ds(sid * RPT, RPT)],
                    deg_out.at[cid, pl.ds(sid * RPT, RPT)])


_deg_call = pl.kernel(
    _deg_body,
    out_type=jax.ShapeDtypeStruct((NC, N_PAD, DEGW), jnp.float32),
    mesh=_mesh,
    scratch_types=[
        pltpu.VMEM((EPW,), jnp.int32),
        pltpu.VMEM((K, DEGW), jnp.float32),
        pltpu.VMEM_SHARED((N_PAD, DEGW), jnp.float32),
        pltpu.SemaphoreType.DMA,
    ],
    compiler_params=pltpu.CompilerParams(use_tc_tiling_on_sc=False),
)


def _scat_body(g_hbm, ei_hbm, s_out,
               src_v, dst_v, rows, s_sh, gsems, ssems):
    cid = lax.axis_index("c")
    sid = lax.axis_index("s")
    wid = sid * NC + cid
    pltpu.async_copy(ei_hbm.at[pl.ds(wid * EPW, EPW)], src_v, gsems.at[1])
    pltpu.async_copy(ei_hbm.at[pl.ds(E + wid * EPW, EPW)], dst_v,
                     gsems.at[1])

    zv = jnp.zeros((32,), jnp.bfloat16)

    def zrow(r, carry):
        for cc in range(D // 32):
            rows[0, r, pl.ds(32 * cc, 32)] = zv
        return carry

    lax.fori_loop(0, K, zrow, 0)
    for t in range(RPT // K):
        pltpu.async_copy(rows.at[0],
                         s_sh.at[pl.ds(sid * RPT + t * K, K)], gsems.at[0])
    for t in range(RPT // K):
        pltpu.make_async_copy(rows.at[0],
                              s_sh.at[pl.ds(sid * RPT, K)], gsems.at[0]).wait()
    pltpu.make_async_copy(ei_hbm.at[pl.ds(wid * EPW, EPW)], src_v,
                          gsems.at[1]).wait()
    pltpu.make_async_copy(ei_hbm.at[pl.ds(E + wid * EPW, EPW)], dst_v,
                          gsems.at[1]).wait()
    plsc.subcore_barrier()

    for b in range(NB):
        pltpu.async_copy(g_hbm.at[src_v.at[pl.ds(b * K, K)]], rows.at[b],
                         gsems.at[b])

    def group(g, carry):
        base = g * NB
        for b in range(NB):
            j = base + b
            pltpu.make_async_copy(g_hbm.at[src_v.at[pl.ds(j * K, K)]],
                                  rows.at[b], gsems.at[b]).wait()
            pltpu.async_copy(rows.at[b], s_sh.at[dst_v.at[pl.ds(j * K, K)]],
                             ssems.at[b], add=True)
        for b in range(NB):
            j2 = base + NB + b

            @pl.when(j2 < C)
            def _(b=b, j2=j2):
                pltpu.make_async_copy(rows.at[b],
                                      s_sh.at[dst_v.at[pl.ds(0, K)]],
                                      ssems.at[b]).wait()
                pltpu.async_copy(g_hbm.at[src_v.at[pl.ds(j2 * K, K)]],
                                 rows.at[b], gsems.at[b])
        return carry

    lax.fori_loop(0, C // NB, group, 0)
    for b in range(NB):
        pltpu.make_async_copy(rows.at[b], s_sh.at[dst_v.at[pl.ds(0, K)]],
                              ssems.at[b]).wait()
    plsc.subcore_barrier()
    pltpu.sync_copy(s_sh.at[pl.ds(sid * RPT, RPT)],
                    s_out.at[cid, pl.ds(sid * RPT, RPT)])


_scat_call = pl.kernel(
    _scat_body,
    out_type=jax.ShapeDtypeStruct((NC, N_PAD, D), jnp.bfloat16),
    mesh=_mesh,
    scratch_types=[
        pltpu.VMEM((EPW,), jnp.int32),
        pltpu.VMEM((EPW,), jnp.int32),
        pltpu.VMEM((NB, K, D), jnp.bfloat16),
        pltpu.VMEM_SHARED((N_PAD, D), jnp.bfloat16),
        pltpu.SemaphoreType.DMA((NB,)),
        pltpu.SemaphoreType.DMA((NB,)),
    ],
    compiler_params=pltpu.CompilerParams(use_tc_tiling_on_sc=False),
)


def _mm_body(x_ref, w_ref, o_ref):
    o_ref[...] = jnp.dot(x_ref[...], w_ref[...],
                         preferred_element_type=jnp.float32)


def _matmul(x, w):
    return pl.pallas_call(
        _mm_body,
        grid=(N // RB,),
        in_specs=[pl.BlockSpec((RB, D), lambda i: (i, 0)),
                  pl.BlockSpec((D, D), lambda i: (0, 0))],
        out_specs=pl.BlockSpec((RB, D), lambda i: (i, 0)),
        out_shape=jax.ShapeDtypeStruct((N, D), jnp.float32),
    )(x, w)


def _dinv_of(deg_ref):
    d0 = deg_ref[0][:, 0:1]
    d1 = deg_ref[1][:, 0:1]
    return lax.rsqrt(d0 + d1 + 1.0)


def _mm_scale_body(deg_ref, x_ref, w_ref, g_ref, gb_ref):
    h = jnp.dot(x_ref[...], w_ref[...], preferred_element_type=jnp.float32)
    g = h * _dinv_of(deg_ref)
    g_ref[...] = g
    gb_ref[...] = g.astype(jnp.bfloat16)


def _mm_scale(deg_parts, x, w):
    return pl.pallas_call(
        _mm_scale_body,
        grid=(N // RB,),
        in_specs=[pl.BlockSpec((NC, RB, DEGW), lambda i: (0, i, 0)),
                  pl.BlockSpec((RB, D), lambda i: (i, 0)),
                  pl.BlockSpec((D, D), lambda i: (0, 0))],
        out_specs=[pl.BlockSpec((RB, D), lambda i: (i, 0)),
                   pl.BlockSpec((RB, D), lambda i: (i, 0))],
        out_shape=[jax.ShapeDtypeStruct((N, D), jnp.float32),
                   jax.ShapeDtypeStruct((N, D), jnp.bfloat16)],
    )(deg_parts, x, w)


def _combine_mm_body(s_ref, g_ref, deg_ref, b_ref, w_ref, o_ref, ob_ref):
    dinv = _dinv_of(deg_ref)
    s = s_ref[0].astype(jnp.float32) + s_ref[1].astype(jnp.float32)
    pre = (s + g_ref[...]) * dinv + b_ref[...]
    act = jnp.maximum(pre, 0.0)
    g2 = jnp.dot(act, w_ref[...],
                 preferred_element_type=jnp.float32) * dinv
    o_ref[...] = g2
    ob_ref[...] = g2.astype(jnp.bfloat16)


def _combine_mm(s_parts, g, deg_parts, b_row, w):
    return pl.pallas_call(
        _combine_mm_body,
        grid=(N // RB,),
        in_specs=[pl.BlockSpec((NC, RB, D), lambda i: (0, i, 0)),
                  pl.BlockSpec((RB, D), lambda i: (i, 0)),
                  pl.BlockSpec((NC, RB, DEGW), lambda i: (0, i, 0)),
                  pl.BlockSpec((1, D), lambda i: (0, 0)),
                  pl.BlockSpec((D, D), lambda i: (0, 0))],
        out_specs=[pl.BlockSpec((RB, D), lambda i: (i, 0)),
                   pl.BlockSpec((RB, D), lambda i: (i, 0))],
        out_shape=[jax.ShapeDtypeStruct((N, D), jnp.float32),
                   jax.ShapeDtypeStruct((N, D), jnp.bfloat16)],
    )(s_parts, g, deg_parts, b_row, w)


def _combine_body(s_ref, g_ref, deg_ref, b_ref, o_ref):
    dinv = _dinv_of(deg_ref)
    s = s_ref[0].astype(jnp.float32) + s_ref[1].astype(jnp.float32)
    pre = (s + g_ref[...]) * dinv + b_ref[...]
    o_ref[...] = jnp.maximum(pre, 0.0)


def _combine(s_parts, g, deg_parts, b_row):
    return pl.pallas_call(
        _combine_body,
        grid=(N // RB,),
        in_specs=[pl.BlockSpec((NC, RB, D), lambda i: (0, i, 0)),
                  pl.BlockSpec((RB, D), lambda i: (i, 0)),
                  pl.BlockSpec((NC, RB, DEGW), lambda i: (0, i, 0)),
                  pl.BlockSpec((1, D), lambda i: (0, 0))],
        out_specs=pl.BlockSpec((RB, D), lambda i: (i, 0)),
        out_shape=jax.ShapeDtypeStruct((N, D), jnp.float32),
    )(s_parts, g, deg_parts, b_row)


def kernel(x, edge_index, W1, b1, W2, b2):
    ei = edge_index.reshape(2 * E)
    ones_deg = jnp.ones((K, DEGW), jnp.float32)
    zeros_deg = jnp.zeros((N_PAD, DEGW), jnp.float32)

    deg_parts = _deg_call(ei, ones_deg, zeros_deg)
    g1, g1b = _mm_scale(deg_parts, x, W1)
    s_parts = _scat_call(g1b, ei)
    g2, g2b = _combine_mm(s_parts, g1, deg_parts, b1.reshape(1, D), W2)
    t_parts = _scat_call(g2b, ei)
    return _combine(t_parts, g2, deg_parts, b2.reshape(1, D))
